# baseline (speedup 1.0000x reference)
import functools

import jax
import jax.numpy as jnp
from jax import lax
from jax.experimental import pallas as pl
from jax.experimental.pallas import tpu as pltpu
from jax.experimental.pallas import tpu_sc as plsc

N = 10000
NP = 10240
E = 320000
D = 128
H = 128

NC = 2
NS = 16
NW = NC * NS
KE = 8000
RB = 1024
EB = 2000


def _sc_gather_rows(table, idx_pad):
    bpw = NP // NW
    mesh = plsc.VectorSubcoreMesh(core_axis_name="c", subcore_axis_name="s")

    @functools.partial(
        pl.kernel,
        mesh=mesh,
        out_type=jax.ShapeDtypeStruct((NP, D), jnp.float32),
        scratch_types=[
            pltpu.VMEM((bpw,), jnp.int32),
            pltpu.VMEM((bpw, D), jnp.float32),
            pltpu.SemaphoreType.DMA,
        ],
    )
    def k(table_hbm, idx_hbm, out_hbm, idx_v, rows_v, sem):
        wid = lax.axis_index("s") * NC + lax.axis_index("c")
        base = wid * bpw
        pltpu.sync_copy(idx_hbm.at[pl.ds(base, bpw)], idx_v)
        copies = []
        for j in range(0, bpw, 80):
            copies.append(
                pltpu.async_copy(
                    table_hbm.at[idx_v.at[pl.ds(j, 80)]],
                    rows_v.at[pl.ds(j, 80)],
                    sem,
                )
            )
        for c in copies:
            c.wait()
        pltpu.sync_copy(rows_v, out_hbm.at[pl.ds(base, bpw)])

    return k(table, idx_pad)


def _sc_edge_aggregate(hmT, sd, ew):
    mesh = plsc.VectorSubcoreMesh(core_axis_name="c", subcore_axis_name="s")

    @functools.partial(
        pl.kernel,
        mesh=mesh,
        compiler_params=pltpu.CompilerParams(
            use_tc_tiling_on_sc=False, needs_layout_passes=False),
        out_type=jax.ShapeDtypeStruct((D + NW, NP), jnp.float32),
        scratch_types=[
            pltpu.VMEM((NP,), jnp.int32),
            pltpu.VMEM((NP,), jnp.int32),
            pltpu.VMEM((NP,), jnp.float32),
            pltpu.VMEM((NP,), jnp.float32),
            pltpu.VMEM((NP,), jnp.float32),
            pltpu.VMEM((NP,), jnp.float32),
            pltpu.VMEM((NP,), jnp.float32),
            pltpu.VMEM((2, KE), jnp.int32),
            pltpu.VMEM((2, KE), jnp.float32),
            pltpu.SemaphoreType.DMA,
            pltpu.SemaphoreType.DMA,
        ],
    )
    def k(hmT_hbm, sd_hbm, ew_hbm, out_hbm,
          c0, c1, a0, a1, a2, a3, ad, pv2, wv2, sem0, sem1):
        cols = (c0, c1)
        accs = (a0, a1, a2, a3)
        wid = lax.axis_index("s") * NC + lax.axis_index("c")
        zeros16 = jnp.broadcast_to((wid * 0).astype(jnp.float32), (16,))

        for r in range(2):
            pltpu.sync_copy(hmT_hbm.at[2 * wid + r], cols[r])

        @plsc.parallel_loop(0, NP, 16, unroll=4)
        def _(b):
            for r in range(4):
                accs[r][pl.ds(b, 16)] = zeros16
            ad[pl.ds(b, 16)] = zeros16

        sems = (sem0, sem1)

        def issue(ch, slot):
            off = ch * KE
            pltpu.async_copy(sd_hbm.at[pl.ds(off, KE)], pv2.at[slot], sems[slot])
            pltpu.async_copy(ew_hbm.at[pl.ds(off, KE)], wv2.at[slot], sems[slot])

        def drain(slot):
            pltpu.make_async_copy(sd_hbm.at[pl.ds(0, KE)], pv2.at[slot], sems[slot]).wait()
            pltpu.make_async_copy(ew_hbm.at[pl.ds(0, KE)], wv2.at[slot], sems[slot]).wait()

        def consume(slot):
            pv, wv = pv2.at[slot], wv2.at[slot]

            @plsc.parallel_loop(0, KE, 16, unroll=16)
            def _(b):
                p = pv[pl.ds(b, 16)]
                s = jnp.bitwise_and(p, 16383)
                d = jnp.right_shift(p, 14)
                w = wv[pl.ds(b, 16)]
                for q in range(2):
                    pk = plsc.load_gather(cols[q], [s])
                    lo = plsc.bitcast(jnp.left_shift(pk, 16), jnp.float32)
                    hi = plsc.bitcast(jnp.bitwise_and(pk, -65536), jnp.float32)
                    plsc.addupdate_scatter(accs[2 * q], [d], lo * w)
                    plsc.addupdate_scatter(accs[2 * q + 1], [d], hi * w)

        npairs = E // (2 * KE)
        issue(0, 0)

        def pair_body(g, _):
            issue(2 * g + 1, 1)
            drain(0)
            consume(0)

            @pl.when(g + 1 < npairs)
            def _():
                issue(2 * g + 2, 0)
            drain(1)
            consume(1)
            return 0
        lax.fori_loop(0, npairs, pair_body, 0)

        shard = E // NW
        pv0, wv0 = pv2.at[0], wv2.at[0]
        for coff, clen in ((0, 8000), (8000, 2000)):
            pltpu.sync_copy(sd_hbm.at[pl.ds(wid * shard + coff, clen)],
                            pv0.at[pl.ds(0, clen)])
            pltpu.sync_copy(ew_hbm.at[pl.ds(wid * shard + coff, clen)],
                            wv0.at[pl.ds(0, clen)])

            @plsc.parallel_loop(0, clen, 16, unroll=8)
            def _(b):
                d = jnp.right_shift(pv0[pl.ds(b, 16)], 14)
                plsc.addupdate_scatter(ad, [d], wv0[pl.ds(b, 16)])

        pltpu.sync_copy(accs[0], out_hbm.at[2 * wid])
        pltpu.sync_copy(accs[1], out_hbm.at[2 * wid + 64])
        pltpu.sync_copy(accs[2], out_hbm.at[2 * wid + 1])
        pltpu.sync_copy(accs[3], out_hbm.at[2 * wid + 65])
        pltpu.sync_copy(ad, out_hbm.at[D + wid])

    return k(hmT, sd, ew)


def _tc_embed_to_messages(x, data2d, W_in, W_msg, src2d, dst2d):
    def body(x_ref, d_ref, wi_ref, wm_ref, s_ref, t_ref, h_ref, hmT_ref, sd_ref):
        m = (d_ref[...] != 0).astype(jnp.float32)
        xb = x_ref[...] * m
        hb = jnp.tanh(jnp.dot(xb, wi_ref[...], preferred_element_type=jnp.float32))
        h_ref[...] = hb
        hm = jnp.dot(hb, wm_ref[...], preferred_element_type=jnp.float32)
        bits = lax.bitcast_convert_type(hm, jnp.int32)
        rne = jnp.right_shift(
            bits + 0x7FFF + jnp.bitwise_and(jnp.right_shift(bits, 16), 1), 16)
        lo = jnp.bitwise_and(rne[:, : H // 2], 0xFFFF)
        hi = jnp.left_shift(rne[:, H // 2 :], 16)
        hmT_ref[...] = jnp.bitwise_or(lo, hi).T
        sd_ref[...] = jnp.bitwise_or(s_ref[...], jnp.left_shift(t_ref[...], 14))

    eb = EB // (NP // RB)
    return pl.pallas_call(
        body,
        grid=(NP // RB,),
        in_specs=[
            pl.BlockSpec((RB, D), lambda j: (j, 0)),
            pl.BlockSpec((RB, 1), lambda j: (j, 0)),
            pl.BlockSpec((D, H), lambda j: (0, 0)),
            pl.BlockSpec((H, H), lambda j: (0, 0)),
            pl.BlockSpec((eb, 160), lambda j: (j, 0)),
            pl.BlockSpec((eb, 160), lambda j: (j, 0)),
        ],
        out_specs=[
            pl.BlockSpec((RB, H), lambda j: (j, 0)),
            pl.BlockSpec((H // 2, RB), lambda j: (0, j)),
            pl.BlockSpec((eb, 160), lambda j: (j, 0)),
        ],
        out_shape=[
            jax.ShapeDtypeStruct((NP, H), jnp.float32),
            jax.ShapeDtypeStruct((H // 2, NP), jnp.int32),
            jax.ShapeDtypeStruct((EB, 160), jnp.int32),
        ],
    )(x, data2d, W_in, W_msg, src2d, dst2d)


def _tc_update(aggT_ext, h, data2d, W_self, b2d):
    def body(a_ref, h_ref, d_ref, ws_ref, b_ref, out_ref, sum_ref):
        j = pl.program_id(0)
        a = a_ref[...]
        deg = jnp.sum(a[D:, :], axis=0)
        agg = a[:D, :].T
        m = (d_ref[...] != 0).astype(jnp.float32)
        h2 = jnp.tanh(
            agg / (deg[:, None] + 1e-6)
            + jnp.dot(h_ref[...], ws_ref[...], preferred_element_type=jnp.float32)
            + b_ref[...]
        )
        out = h2 * m
        out_ref[...] = out
        ps = jnp.sum(out, axis=0, keepdims=True)

        @pl.when(j == 0)
        def _():
            sum_ref[...] = ps

        @pl.when(j != 0)
        def _():
            sum_ref[...] += ps

    return pl.pallas_call(
        body,
        grid=(NP // RB,),
        in_specs=[
            pl.BlockSpec((D + NW, RB), lambda j: (0, j)),
            pl.BlockSpec((RB, H), lambda j: (j, 0)),
            pl.BlockSpec((RB, 1), lambda j: (j, 0)),
            pl.BlockSpec((H, H), lambda j: (0, 0)),
            pl.BlockSpec((1, H), lambda j: (0, 0)),
        ],
        out_specs=[
            pl.BlockSpec((RB, H), lambda j: (j, 0)),
            pl.BlockSpec((1, H), lambda j: (0, 0)),
        ],
        out_shape=[
            jax.ShapeDtypeStruct((NP, H), jnp.float32),
            jax.ShapeDtypeStruct((1, H), jnp.float32),
        ],
    )(aggT_ext, h, data2d, W_self, b2d)


def kernel(data, edge_index, edge_weight, embed, W_in, W_msg, W_self, b):
    data_pad = jnp.pad(data.astype(jnp.int32), (0, NP - N))
    data2d = data_pad.reshape(NP, 1)

    ei3 = edge_index.astype(jnp.int32).reshape(2, EB, 160)
    x = _sc_gather_rows(embed, data_pad)
    h, hmT, sd2d = _tc_embed_to_messages(x, data2d, W_in, W_msg, ei3[0], ei3[1])

    aggT_ext = _sc_edge_aggregate(hmT, sd2d.reshape(E), edge_weight)

    out_pad, sum_vec = _tc_update(aggT_ext, h, data2d, W_self, b.reshape(1, H))

    denom = jnp.maximum(jnp.sum(data != 0).astype(jnp.float32), 1.0)
    return (out_pad[:N], sum_vec[0] / denom)


# speedup vs baseline: 1.0132x; 1.0132x over previous
import functools

import jax
import jax.numpy as jnp
from jax import lax
from jax.experimental import pallas as pl
from jax.experimental.pallas import tpu as pltpu
from jax.experimental.pallas import tpu_sc as plsc

N = 10000
NP = 10240
E = 320000
D = 128
H = 128

NC = 2
NS = 16
NW = NC * NS
KE = 8000
RB = 1024
EB = 2000


def _sc_gather_rows(table, idx_pad):
    bpw = NP // NW
    mesh = plsc.VectorSubcoreMesh(core_axis_name="c", subcore_axis_name="s")

    @functools.partial(
        pl.kernel,
        mesh=mesh,
        out_type=jax.ShapeDtypeStruct((NP, D), jnp.float32),
        scratch_types=[
            pltpu.VMEM((bpw,), jnp.int32),
            pltpu.VMEM((bpw, D), jnp.float32),
            pltpu.SemaphoreType.DMA,
            pltpu.SemaphoreType.DMA,
        ],
    )
    def k(table_hbm, idx_hbm, out_hbm, idx_v, rows_v, gsem, osem):
        wid = lax.axis_index("s") * NC + lax.axis_index("c")
        base = wid * bpw
        pltpu.sync_copy(idx_hbm.at[pl.ds(base, bpw)], idx_v)
        gathers = []
        for j in range(0, bpw, 80):
            gathers.append(
                pltpu.async_copy(
                    table_hbm.at[idx_v.at[pl.ds(j, 80)]],
                    rows_v.at[pl.ds(j, 80)],
                    gsem,
                )
            )
        outs = []
        for i, j in enumerate(range(0, bpw, 80)):
            gathers[i].wait()
            outs.append(
                pltpu.async_copy(
                    rows_v.at[pl.ds(j, 80)],
                    out_hbm.at[pl.ds(base + j, 80)],
                    osem,
                )
            )
        for c in outs:
            c.wait()

    return k(table, idx_pad)


def _sc_edge_aggregate(hmT, sd, ew):
    mesh = plsc.VectorSubcoreMesh(core_axis_name="c", subcore_axis_name="s")

    @functools.partial(
        pl.kernel,
        mesh=mesh,
        compiler_params=pltpu.CompilerParams(
            use_tc_tiling_on_sc=False, needs_layout_passes=False),
        out_type=jax.ShapeDtypeStruct((D + NW, NP), jnp.float32),
        scratch_types=[
            pltpu.VMEM((NP,), jnp.int32),
            pltpu.VMEM((NP,), jnp.int32),
            pltpu.VMEM((NP,), jnp.float32),
            pltpu.VMEM((NP,), jnp.float32),
            pltpu.VMEM((NP,), jnp.float32),
            pltpu.VMEM((NP,), jnp.float32),
            pltpu.VMEM((NP,), jnp.float32),
            pltpu.VMEM((2, KE), jnp.int32),
            pltpu.VMEM((2, KE), jnp.float32),
            pltpu.SemaphoreType.DMA,
            pltpu.SemaphoreType.DMA,
        ],
    )
    def k(hmT_hbm, sd_hbm, ew_hbm, out_hbm,
          c0, c1, a0, a1, a2, a3, ad, pv2, wv2, sem0, sem1):
        cols = (c0, c1)
        accs = (a0, a1, a2, a3)
        wid = lax.axis_index("s") * NC + lax.axis_index("c")
        zeros16 = jnp.broadcast_to((wid * 0).astype(jnp.float32), (16,))

        for r in range(2):
            pltpu.sync_copy(hmT_hbm.at[2 * wid + r], cols[r])

        @plsc.parallel_loop(0, NP, 16, unroll=4)
        def _(b):
            for r in range(4):
                accs[r][pl.ds(b, 16)] = zeros16
            ad[pl.ds(b, 16)] = zeros16

        sems = (sem0, sem1)

        def issue(ch, slot):
            off = ch * KE
            pltpu.async_copy(sd_hbm.at[pl.ds(off, KE)], pv2.at[slot], sems[slot])
            pltpu.async_copy(ew_hbm.at[pl.ds(off, KE)], wv2.at[slot], sems[slot])

        def drain(slot):
            pltpu.make_async_copy(sd_hbm.at[pl.ds(0, KE)], pv2.at[slot], sems[slot]).wait()
            pltpu.make_async_copy(ew_hbm.at[pl.ds(0, KE)], wv2.at[slot], sems[slot]).wait()

        def consume(slot):
            pv, wv = pv2.at[slot], wv2.at[slot]

            @plsc.parallel_loop(0, KE, 16, unroll=8)
            def _(b):
                p = pv[pl.ds(b, 16)]
                s = jnp.bitwise_and(p, 16383)
                d = jnp.right_shift(p, 14)
                w = wv[pl.ds(b, 16)]
                for q in range(2):
                    pk = plsc.load_gather(cols[q], [s])
                    lo = plsc.bitcast(jnp.left_shift(pk, 16), jnp.float32)
                    hi = plsc.bitcast(jnp.bitwise_and(pk, -65536), jnp.float32)
                    plsc.addupdate_scatter(accs[2 * q], [d], lo * w)
                    plsc.addupdate_scatter(accs[2 * q + 1], [d], hi * w)

        npairs = E // (2 * KE)
        issue(0, 0)

        def pair_body(g, _):
            issue(2 * g + 1, 1)
            drain(0)
            consume(0)

            @pl.when(g + 1 < npairs)
            def _():
                issue(2 * g + 2, 0)
            drain(1)
            consume(1)
            return 0
        lax.fori_loop(0, npairs, pair_body, 0)

        shard = E // NW
        pv0, wv0 = pv2.at[0], wv2.at[0]
        for coff, clen in ((0, 8000), (8000, 2000)):
            pltpu.sync_copy(sd_hbm.at[pl.ds(wid * shard + coff, clen)],
                            pv0.at[pl.ds(0, clen)])
            pltpu.sync_copy(ew_hbm.at[pl.ds(wid * shard + coff, clen)],
                            wv0.at[pl.ds(0, clen)])

            @plsc.parallel_loop(0, clen, 16, unroll=8)
            def _(b):
                d = jnp.right_shift(pv0[pl.ds(b, 16)], 14)
                plsc.addupdate_scatter(ad, [d], wv0[pl.ds(b, 16)])

        pltpu.sync_copy(accs[0], out_hbm.at[2 * wid])
        pltpu.sync_copy(accs[1], out_hbm.at[2 * wid + 64])
        pltpu.sync_copy(accs[2], out_hbm.at[2 * wid + 1])
        pltpu.sync_copy(accs[3], out_hbm.at[2 * wid + 65])
        pltpu.sync_copy(ad, out_hbm.at[D + wid])

    return k(hmT, sd, ew)


def _tc_embed_to_messages(x, data2d, W_in, W_msg, src2d, dst2d):
    def body(x_ref, d_ref, wi_ref, wm_ref, s_ref, t_ref, h_ref, hmT_ref, sd_ref):
        m = (d_ref[...] != 0).astype(jnp.float32)
        xb = x_ref[...] * m
        hb = jnp.tanh(jnp.dot(xb, wi_ref[...], preferred_element_type=jnp.float32))
        h_ref[...] = hb
        hm = jnp.dot(hb, wm_ref[...], preferred_element_type=jnp.float32)
        bits = lax.bitcast_convert_type(hm, jnp.int32)
        rne = jnp.right_shift(
            bits + 0x7FFF + jnp.bitwise_and(jnp.right_shift(bits, 16), 1), 16)
        lo = jnp.bitwise_and(rne[:, : H // 2], 0xFFFF)
        hi = jnp.left_shift(rne[:, H // 2 :], 16)
        hmT_ref[...] = jnp.bitwise_or(lo, hi).T
        sd_ref[...] = jnp.bitwise_or(s_ref[...], jnp.left_shift(t_ref[...], 14))

    eb = EB // (NP // RB)
    return pl.pallas_call(
        body,
        grid=(NP // RB,),
        in_specs=[
            pl.BlockSpec((RB, D), lambda j: (j, 0)),
            pl.BlockSpec((RB, 1), lambda j: (j, 0)),
            pl.BlockSpec((D, H), lambda j: (0, 0)),
            pl.BlockSpec((H, H), lambda j: (0, 0)),
            pl.BlockSpec((eb, 160), lambda j: (j, 0)),
            pl.BlockSpec((eb, 160), lambda j: (j, 0)),
        ],
        out_specs=[
            pl.BlockSpec((RB, H), lambda j: (j, 0)),
            pl.BlockSpec((H // 2, RB), lambda j: (0, j)),
            pl.BlockSpec((eb, 160), lambda j: (j, 0)),
        ],
        out_shape=[
            jax.ShapeDtypeStruct((NP, H), jnp.float32),
            jax.ShapeDtypeStruct((H // 2, NP), jnp.int32),
            jax.ShapeDtypeStruct((EB, 160), jnp.int32),
        ],
    )(x, data2d, W_in, W_msg, src2d, dst2d)


def _tc_update(aggT_ext, h, data2d, W_self, b2d):
    def body(a_ref, h_ref, d_ref, ws_ref, b_ref, out_ref, sum_ref):
        j = pl.program_id(0)
        a = a_ref[...]
        deg = jnp.sum(a[D:, :], axis=0)
        agg = a[:D, :].T
        m = (d_ref[...] != 0).astype(jnp.float32)
        h2 = jnp.tanh(
            agg / (deg[:, None] + 1e-6)
            + jnp.dot(h_ref[...], ws_ref[...], preferred_element_type=jnp.float32)
            + b_ref[...]
        )
        out = h2 * m
        out_ref[...] = out
        ps = jnp.sum(out, axis=0, keepdims=True)

        @pl.when(j == 0)
        def _():
            sum_ref[...] = ps

        @pl.when(j != 0)
        def _():
            sum_ref[...] += ps

    return pl.pallas_call(
        body,
        grid=(NP // RB,),
        in_specs=[
            pl.BlockSpec((D + NW, RB), lambda j: (0, j)),
            pl.BlockSpec((RB, H), lambda j: (j, 0)),
            pl.BlockSpec((RB, 1), lambda j: (j, 0)),
            pl.BlockSpec((H, H), lambda j: (0, 0)),
            pl.BlockSpec((1, H), lambda j: (0, 0)),
        ],
        out_specs=[
            pl.BlockSpec((RB, H), lambda j: (j, 0)),
            pl.BlockSpec((1, H), lambda j: (0, 0)),
        ],
        out_shape=[
            jax.ShapeDtypeStruct((NP, H), jnp.float32),
            jax.ShapeDtypeStruct((1, H), jnp.float32),
        ],
    )(aggT_ext, h, data2d, W_self, b2d)


def kernel(data, edge_index, edge_weight, embed, W_in, W_msg, W_self, b):
    data_pad = jnp.pad(data.astype(jnp.int32), (0, NP - N))
    data2d = data_pad.reshape(NP, 1)

    ei3 = edge_index.astype(jnp.int32).reshape(2, EB, 160)
    x = _sc_gather_rows(embed, data_pad)
    h, hmT, sd2d = _tc_embed_to_messages(x, data2d, W_in, W_msg, ei3[0], ei3[1])

    aggT_ext = _sc_edge_aggregate(hmT, sd2d.reshape(E), edge_weight)

    out_pad, sum_vec = _tc_update(aggT_ext, h, data2d, W_self, b.reshape(1, H))

    denom = jnp.maximum(jnp.sum(data != 0).astype(jnp.float32), 1.0)
    return (out_pad[:N], sum_vec[0] / denom)


# speedup vs baseline: 1.0272x; 1.0138x over previous
import functools

import jax
import jax.numpy as jnp
from jax import lax
from jax.experimental import pallas as pl
from jax.experimental.pallas import tpu as pltpu
from jax.experimental.pallas import tpu_sc as plsc

N = 10000
NP = 10240
E = 320000
D = 128
H = 128

NC = 2
NS = 16
NW = NC * NS
KE = 10000
RB = 1024
EB = 2000


def _sc_gather_rows(table, idx_pad):
    bpw = NP // NW
    mesh = plsc.VectorSubcoreMesh(core_axis_name="c", subcore_axis_name="s")

    @functools.partial(
        pl.kernel,
        mesh=mesh,
        out_type=jax.ShapeDtypeStruct((NP, D), jnp.float32),
        scratch_types=[
            pltpu.VMEM((bpw,), jnp.int32),
            pltpu.VMEM((bpw, D), jnp.float32),
            pltpu.SemaphoreType.DMA,
        ],
    )
    def k(table_hbm, idx_hbm, out_hbm, idx_v, rows_v, sem):
        wid = lax.axis_index("s") * NC + lax.axis_index("c")
        base = wid * bpw
        pltpu.sync_copy(idx_hbm.at[pl.ds(base, bpw)], idx_v)
        copies = []
        for j in range(0, bpw, 80):
            copies.append(
                pltpu.async_copy(
                    table_hbm.at[idx_v.at[pl.ds(j, 80)]],
                    rows_v.at[pl.ds(j, 80)],
                    sem,
                )
            )
        for c in copies:
            c.wait()
        pltpu.sync_copy(rows_v, out_hbm.at[pl.ds(base, bpw)])

    return k(table, idx_pad)


def _sc_edge_aggregate(hmT, sd, ew):
    mesh = plsc.VectorSubcoreMesh(core_axis_name="c", subcore_axis_name="s")

    @functools.partial(
        pl.kernel,
        mesh=mesh,
        compiler_params=pltpu.CompilerParams(
            use_tc_tiling_on_sc=False, needs_layout_passes=False),
        out_type=jax.ShapeDtypeStruct((D + NW, NP), jnp.float32),
        scratch_types=[
            pltpu.VMEM((NP,), jnp.int32),
            pltpu.VMEM((NP,), jnp.int32),
            pltpu.VMEM((NP,), jnp.float32),
            pltpu.VMEM((NP,), jnp.float32),
            pltpu.VMEM((NP,), jnp.float32),
            pltpu.VMEM((NP,), jnp.float32),
            pltpu.VMEM((NP,), jnp.float32),
            pltpu.VMEM((2, KE), jnp.int32),
            pltpu.VMEM((2, KE), jnp.float32),
            pltpu.SemaphoreType.DMA,
            pltpu.SemaphoreType.DMA,
        ],
    )
    def k(hmT_hbm, sd_hbm, ew_hbm, out_hbm,
          c0, c1, a0, a1, a2, a3, ad, pv2, wv2, sem0, sem1):
        cols = (c0, c1)
        accs = (a0, a1, a2, a3)
        wid = lax.axis_index("s") * NC + lax.axis_index("c")
        zeros16 = jnp.broadcast_to((wid * 0).astype(jnp.float32), (16,))

        for r in range(2):
            pltpu.sync_copy(hmT_hbm.at[2 * wid + r], cols[r])

        @plsc.parallel_loop(0, NP, 16, unroll=4)
        def _(b):
            for r in range(4):
                accs[r][pl.ds(b, 16)] = zeros16
            ad[pl.ds(b, 16)] = zeros16

        sems = (sem0, sem1)

        def issue(ch, slot):
            off = ch * KE
            pltpu.async_copy(sd_hbm.at[pl.ds(off, KE)], pv2.at[slot], sems[slot])
            pltpu.async_copy(ew_hbm.at[pl.ds(off, KE)], wv2.at[slot], sems[slot])

        def drain(slot):
            pltpu.make_async_copy(sd_hbm.at[pl.ds(0, KE)], pv2.at[slot], sems[slot]).wait()
            pltpu.make_async_copy(ew_hbm.at[pl.ds(0, KE)], wv2.at[slot], sems[slot]).wait()

        def consume(slot):
            pv, wv = pv2.at[slot], wv2.at[slot]

            @plsc.parallel_loop(0, KE, 16, unroll=8)
            def _(b):
                p = pv[pl.ds(b, 16)]
                s = jnp.bitwise_and(p, 16383)
                d = jnp.right_shift(p, 14)
                w = wv[pl.ds(b, 16)]
                for q in range(2):
                    pk = plsc.load_gather(cols[q], [s])
                    lo = plsc.bitcast(jnp.left_shift(pk, 16), jnp.float32)
                    hi = plsc.bitcast(jnp.bitwise_and(pk, -65536), jnp.float32)
                    plsc.addupdate_scatter(accs[2 * q], [d], lo * w)
                    plsc.addupdate_scatter(accs[2 * q + 1], [d], hi * w)

        npairs = E // (2 * KE)
        issue(0, 0)

        def pair_body(g, _):
            issue(2 * g + 1, 1)
            drain(0)
            consume(0)

            @pl.when(g + 1 < npairs)
            def _():
                issue(2 * g + 2, 0)
            drain(1)
            consume(1)
            return 0
        lax.fori_loop(0, npairs, pair_body, 0)

        shard = E // NW
        pv0, wv0 = pv2.at[0], wv2.at[0]
        for coff, clen in ((0, 10000),):
            pltpu.sync_copy(sd_hbm.at[pl.ds(wid * shard + coff, clen)],
                            pv0.at[pl.ds(0, clen)])
            pltpu.sync_copy(ew_hbm.at[pl.ds(wid * shard + coff, clen)],
                            wv0.at[pl.ds(0, clen)])

            @plsc.parallel_loop(0, clen, 16, unroll=8)
            def _(b):
                d = jnp.right_shift(pv0[pl.ds(b, 16)], 14)
                plsc.addupdate_scatter(ad, [d], wv0[pl.ds(b, 16)])

        pltpu.sync_copy(accs[0], out_hbm.at[2 * wid])
        pltpu.sync_copy(accs[1], out_hbm.at[2 * wid + 64])
        pltpu.sync_copy(accs[2], out_hbm.at[2 * wid + 1])
        pltpu.sync_copy(accs[3], out_hbm.at[2 * wid + 65])
        pltpu.sync_copy(ad, out_hbm.at[D + wid])

    return k(hmT, sd, ew)


def _tc_embed_to_messages(x, data2d, W_in, W_msg, src2d, dst2d):
    def body(x_ref, d_ref, wi_ref, wm_ref, s_ref, t_ref, h_ref, hmT_ref, sd_ref):
        m = (d_ref[...] != 0).astype(jnp.float32)
        xb = x_ref[...] * m
        hb = jnp.tanh(jnp.dot(xb, wi_ref[...], preferred_element_type=jnp.float32))
        h_ref[...] = hb
        hm = jnp.dot(hb, wm_ref[...], preferred_element_type=jnp.float32)
        bits = lax.bitcast_convert_type(hm, jnp.int32)
        rne = jnp.right_shift(
            bits + 0x7FFF + jnp.bitwise_and(jnp.right_shift(bits, 16), 1), 16)
        lo = jnp.bitwise_and(rne[:, : H // 2], 0xFFFF)
        hi = jnp.left_shift(rne[:, H // 2 :], 16)
        hmT_ref[...] = jnp.bitwise_or(lo, hi).T
        sd_ref[...] = jnp.bitwise_or(s_ref[...], jnp.left_shift(t_ref[...], 14))

    eb = EB // (NP // RB)
    return pl.pallas_call(
        body,
        grid=(NP // RB,),
        in_specs=[
            pl.BlockSpec((RB, D), lambda j: (j, 0)),
            pl.BlockSpec((RB, 1), lambda j: (j, 0)),
            pl.BlockSpec((D, H), lambda j: (0, 0)),
            pl.BlockSpec((H, H), lambda j: (0, 0)),
            pl.BlockSpec((eb, 160), lambda j: (j, 0)),
            pl.BlockSpec((eb, 160), lambda j: (j, 0)),
        ],
        out_specs=[
            pl.BlockSpec((RB, H), lambda j: (j, 0)),
            pl.BlockSpec((H // 2, RB), lambda j: (0, j)),
            pl.BlockSpec((eb, 160), lambda j: (j, 0)),
        ],
        out_shape=[
            jax.ShapeDtypeStruct((NP, H), jnp.float32),
            jax.ShapeDtypeStruct((H // 2, NP), jnp.int32),
            jax.ShapeDtypeStruct((EB, 160), jnp.int32),
        ],
    )(x, data2d, W_in, W_msg, src2d, dst2d)


def _tc_update(aggT_ext, h, data2d, W_self, b2d):
    def body(a_ref, h_ref, d_ref, ws_ref, b_ref, out_ref, sum_ref):
        j = pl.program_id(0)
        a = a_ref[...]
        deg = jnp.sum(a[D:, :], axis=0)
        agg = a[:D, :].T
        m = (d_ref[...] != 0).astype(jnp.float32)
        h2 = jnp.tanh(
            agg / (deg[:, None] + 1e-6)
            + jnp.dot(h_ref[...], ws_ref[...], preferred_element_type=jnp.float32)
            + b_ref[...]
        )
        out = h2 * m
        out_ref[...] = out
        ps = jnp.sum(out, axis=0, keepdims=True)

        @pl.when(j == 0)
        def _():
            sum_ref[...] = ps

        @pl.when(j != 0)
        def _():
            sum_ref[...] += ps

    return pl.pallas_call(
        body,
        grid=(NP // RB,),
        in_specs=[
            pl.BlockSpec((D + NW, RB), lambda j: (0, j)),
            pl.BlockSpec((RB, H), lambda j: (j, 0)),
            pl.BlockSpec((RB, 1), lambda j: (j, 0)),
            pl.BlockSpec((H, H), lambda j: (0, 0)),
            pl.BlockSpec((1, H), lambda j: (0, 0)),
        ],
        out_specs=[
            pl.BlockSpec((RB, H), lambda j: (j, 0)),
            pl.BlockSpec((1, H), lambda j: (0, 0)),
        ],
        out_shape=[
            jax.ShapeDtypeStruct((NP, H), jnp.float32),
            jax.ShapeDtypeStruct((1, H), jnp.float32),
        ],
    )(aggT_ext, h, data2d, W_self, b2d)


def kernel(data, edge_index, edge_weight, embed, W_in, W_msg, W_self, b):
    data_pad = jnp.pad(data.astype(jnp.int32), (0, NP - N))
    data2d = data_pad.reshape(NP, 1)

    ei3 = edge_index.astype(jnp.int32).reshape(2, EB, 160)
    x = _sc_gather_rows(embed, data_pad)
    h, hmT, sd2d = _tc_embed_to_messages(x, data2d, W_in, W_msg, ei3[0], ei3[1])

    aggT_ext = _sc_edge_aggregate(hmT, sd2d.reshape(E), edge_weight)

    out_pad, sum_vec = _tc_update(aggT_ext, h, data2d, W_self, b.reshape(1, H))

    denom = jnp.maximum(jnp.sum(data != 0).astype(jnp.float32), 1.0)
    return (out_pad[:N], sum_vec[0] / denom)


# speedup vs baseline: 1.0481x; 1.0204x over previous
import functools

import jax
import jax.numpy as jnp
from jax import lax
from jax.experimental import pallas as pl
from jax.experimental.pallas import tpu as pltpu
from jax.experimental.pallas import tpu_sc as plsc

N = 10000
NP = 10240
E = 320000
D = 128
H = 128

NC = 2
NS = 16
NW = NC * NS
KE = 10000
RB = 1024
EB = 2000


def _sc_gather_rows(table, idx_pad):
    bpw = NP // NW
    mesh = plsc.VectorSubcoreMesh(core_axis_name="c", subcore_axis_name="s")

    @functools.partial(
        pl.kernel,
        mesh=mesh,
        out_type=jax.ShapeDtypeStruct((NP, D), jnp.float32),
        scratch_types=[
            pltpu.VMEM((bpw,), jnp.int32),
            pltpu.VMEM((bpw, D), jnp.float32),
            pltpu.SemaphoreType.DMA,
        ],
    )
    def k(table_hbm, idx_hbm, out_hbm, idx_v, rows_v, sem):
        wid = lax.axis_index("s") * NC + lax.axis_index("c")
        base = wid * bpw
        pltpu.sync_copy(idx_hbm.at[pl.ds(base, bpw)], idx_v)
        copies = []
        for j in range(0, bpw, 80):
            copies.append(
                pltpu.async_copy(
                    table_hbm.at[idx_v.at[pl.ds(j, 80)]],
                    rows_v.at[pl.ds(j, 80)],
                    sem,
                )
            )
        for c in copies:
            c.wait()
        pltpu.sync_copy(rows_v, out_hbm.at[pl.ds(base, bpw)])

    return k(table, idx_pad)


def _sc_edge_aggregate(hmT, sd, ew):
    mesh = plsc.VectorSubcoreMesh(core_axis_name="c", subcore_axis_name="s")

    @functools.partial(
        pl.kernel,
        mesh=mesh,
        compiler_params=pltpu.CompilerParams(
            use_tc_tiling_on_sc=False, needs_layout_passes=False),
        out_type=jax.ShapeDtypeStruct((D + NW, NP), jnp.float32),
        scratch_types=[
            pltpu.VMEM((NP,), jnp.int32),
            pltpu.VMEM((NP,), jnp.int32),
            pltpu.VMEM((NP,), jnp.float32),
            pltpu.VMEM((NP,), jnp.float32),
            pltpu.VMEM((NP,), jnp.float32),
            pltpu.VMEM((NP,), jnp.float32),
            pltpu.VMEM((NP,), jnp.float32),
            pltpu.VMEM((2, KE), jnp.int32),
            pltpu.VMEM((2, KE), jnp.float32),
            pltpu.SemaphoreType.DMA,
            pltpu.SemaphoreType.DMA,
        ],
    )
    def k(hmT_hbm, sd_hbm, ew_hbm, out_hbm,
          c0, c1, a0, a1, a2, a3, ad, pv2, wv2, sem0, sem1):
        cols = (c0, c1)
        accs = (a0, a1, a2, a3)
        wid = lax.axis_index("s") * NC + lax.axis_index("c")
        zeros16 = jnp.broadcast_to((wid * 0).astype(jnp.float32), (16,))

        for r in range(2):
            pltpu.sync_copy(hmT_hbm.at[2 * wid + r], cols[r])

        @plsc.parallel_loop(0, NP, 16, unroll=4)
        def _(b):
            for r in range(4):
                accs[r][pl.ds(b, 16)] = zeros16
            ad[pl.ds(b, 16)] = zeros16

        sems = (sem0, sem1)

        def issue(ch, slot):
            off = ch * KE
            pltpu.async_copy(sd_hbm.at[pl.ds(off, KE)], pv2.at[slot], sems[slot])
            pltpu.async_copy(ew_hbm.at[pl.ds(off, KE)], wv2.at[slot], sems[slot])

        def drain(slot):
            pltpu.make_async_copy(sd_hbm.at[pl.ds(0, KE)], pv2.at[slot], sems[slot]).wait()
            pltpu.make_async_copy(ew_hbm.at[pl.ds(0, KE)], wv2.at[slot], sems[slot]).wait()

        def consume(slot):
            pv, wv = pv2.at[slot], wv2.at[slot]

            @plsc.parallel_loop(0, KE, 16, unroll=8)
            def _(b):
                p = pv[pl.ds(b, 16)]
                s = jnp.bitwise_and(p, 16383)
                d = jnp.right_shift(p, 14)
                w = wv[pl.ds(b, 16)]
                for q in range(2):
                    pk = plsc.load_gather(cols[q], [s])
                    lo = plsc.bitcast(jnp.left_shift(pk, 16), jnp.float32)
                    hi = plsc.bitcast(jnp.bitwise_and(pk, -65536), jnp.float32)
                    plsc.addupdate_scatter(accs[2 * q], [d], lo * w)
                    plsc.addupdate_scatter(accs[2 * q + 1], [d], hi * w)

        npairs = E // (2 * KE)
        issue(0, 0)

        def pair_body(g, _):
            issue(2 * g + 1, 1)
            drain(0)
            consume(0)

            @pl.when(g + 1 < npairs)
            def _():
                issue(2 * g + 2, 0)
            drain(1)
            consume(1)
            return 0
        lax.fori_loop(0, npairs, pair_body, 0)

        shard = E // NW
        pv0, wv0 = pv2.at[0], wv2.at[0]
        for coff, clen in ((0, 10000),):
            pltpu.sync_copy(sd_hbm.at[pl.ds(wid * shard + coff, clen)],
                            pv0.at[pl.ds(0, clen)])
            pltpu.sync_copy(ew_hbm.at[pl.ds(wid * shard + coff, clen)],
                            wv0.at[pl.ds(0, clen)])

            @plsc.parallel_loop(0, clen, 16, unroll=8)
            def _(b):
                d = jnp.right_shift(pv0[pl.ds(b, 16)], 14)
                plsc.addupdate_scatter(ad, [d], wv0[pl.ds(b, 16)])

        pltpu.sync_copy(accs[0], out_hbm.at[2 * wid])
        pltpu.sync_copy(accs[1], out_hbm.at[2 * wid + 64])
        pltpu.sync_copy(accs[2], out_hbm.at[2 * wid + 1])
        pltpu.sync_copy(accs[3], out_hbm.at[2 * wid + 65])
        pltpu.sync_copy(ad, out_hbm.at[D + wid])

    return k(hmT, sd, ew)


def _tc_embed_to_messages(x, data2d, W_in, W_msg, src2d, dst2d):
    def body(x_ref, d_ref, wi_ref, wm_ref, s_ref, t_ref, h_ref, hmT_ref, sd_ref):
        m = (d_ref[...] != 0).astype(jnp.float32)
        xb = x_ref[...] * m
        hb = jnp.tanh(jnp.dot(xb, wi_ref[...], preferred_element_type=jnp.float32))
        h_ref[...] = hb
        hm = jnp.dot(hb, wm_ref[...], preferred_element_type=jnp.float32)
        bits = lax.bitcast_convert_type(hm, jnp.int32)
        rne = jnp.right_shift(
            bits + 0x7FFF + jnp.bitwise_and(jnp.right_shift(bits, 16), 1), 16)
        lo = jnp.bitwise_and(rne[:, : H // 2], 0xFFFF)
        hi = jnp.left_shift(rne[:, H // 2 :], 16)
        hmT_ref[...] = jnp.bitwise_or(lo, hi).T
        sd_ref[...] = jnp.bitwise_or(s_ref[...], jnp.left_shift(t_ref[...], 14))

    eb = EB // (NP // RB)
    return pl.pallas_call(
        body,
        grid=(NP // RB,),
        in_specs=[
            pl.BlockSpec((RB, D), lambda j: (j, 0)),
            pl.BlockSpec((RB, 1), lambda j: (j, 0)),
            pl.BlockSpec((D, H), lambda j: (0, 0)),
            pl.BlockSpec((H, H), lambda j: (0, 0)),
            pl.BlockSpec((eb, 160), lambda j: (j, 0)),
            pl.BlockSpec((eb, 160), lambda j: (j, 0)),
        ],
        out_specs=[
            pl.BlockSpec((RB, H), lambda j: (j, 0)),
            pl.BlockSpec((H // 2, RB), lambda j: (0, j)),
            pl.BlockSpec((eb, 160), lambda j: (j, 0)),
        ],
        out_shape=[
            jax.ShapeDtypeStruct((NP, H), jnp.float32),
            jax.ShapeDtypeStruct((H // 2, NP), jnp.int32),
            jax.ShapeDtypeStruct((EB, 160), jnp.int32),
        ],
    )(x, data2d, W_in, W_msg, src2d, dst2d)


def _tc_update(aggT_ext, h, data2d, W_self, b2d):
    def body(a_ref, h_ref, d_ref, ws_ref, b_ref, out_ref, sum_ref, cnt_ref):
        j = pl.program_id(0)
        a = a_ref[...]
        deg = jnp.sum(a[D:, :], axis=0)
        agg = a[:D, :].T
        m = (d_ref[...] != 0).astype(jnp.float32)
        h2 = jnp.tanh(
            agg / (deg[:, None] + 1e-6)
            + jnp.dot(h_ref[...], ws_ref[...], preferred_element_type=jnp.float32)
            + b_ref[...]
        )
        out = h2 * m
        out_ref[...] = out
        ps = jnp.sum(out, axis=0, keepdims=True)
        pc = jnp.sum(m, keepdims=True).reshape(1, 1)

        @pl.when(j == 0)
        def _():
            sum_ref[...] = ps
            cnt_ref[...] = pc

        @pl.when(j != 0)
        def _():
            sum_ref[...] += ps
            cnt_ref[...] += pc

    return pl.pallas_call(
        body,
        grid=(NP // RB,),
        in_specs=[
            pl.BlockSpec((D + NW, RB), lambda j: (0, j)),
            pl.BlockSpec((RB, H), lambda j: (j, 0)),
            pl.BlockSpec((RB, 1), lambda j: (j, 0)),
            pl.BlockSpec((H, H), lambda j: (0, 0)),
            pl.BlockSpec((1, H), lambda j: (0, 0)),
        ],
        out_specs=[
            pl.BlockSpec((RB, H), lambda j: (j, 0)),
            pl.BlockSpec((1, H), lambda j: (0, 0)),
            pl.BlockSpec((1, 1), lambda j: (0, 0)),
        ],
        out_shape=[
            jax.ShapeDtypeStruct((N, H), jnp.float32),
            jax.ShapeDtypeStruct((1, H), jnp.float32),
            jax.ShapeDtypeStruct((1, 1), jnp.float32),
        ],
    )(aggT_ext, h, data2d, W_self, b2d)


def kernel(data, edge_index, edge_weight, embed, W_in, W_msg, W_self, b):
    data_pad = jnp.pad(data.astype(jnp.int32), (0, NP - N))
    data2d = data_pad.reshape(NP, 1)

    ei3 = edge_index.astype(jnp.int32).reshape(2, EB, 160)
    x = _sc_gather_rows(embed, data_pad)
    h, hmT, sd2d = _tc_embed_to_messages(x, data2d, W_in, W_msg, ei3[0], ei3[1])

    aggT_ext = _sc_edge_aggregate(hmT, sd2d.reshape(E), edge_weight)

    out, sum_vec, cnt = _tc_update(aggT_ext, h, data2d, W_self, b.reshape(1, H))

    denom = jnp.maximum(cnt[0, 0], 1.0)
    return (out, sum_vec[0] / denom)


# speedup vs baseline: 1.0491x; 1.0010x over previous
import functools

import jax
import jax.numpy as jnp
from jax import lax
from jax.experimental import pallas as pl
from jax.experimental.pallas import tpu as pltpu
from jax.experimental.pallas import tpu_sc as plsc

N = 10000
NP = 10240
E = 320000
D = 128
H = 128

NC = 2
NS = 16
NW = NC * NS
KE = 10000
RB = 1024
EB = 2000


def _sc_gather_rows(table, idx_pad):
    bpw = NP // NW
    mesh = plsc.VectorSubcoreMesh(core_axis_name="c", subcore_axis_name="s")

    @functools.partial(
        pl.kernel,
        mesh=mesh,
        out_type=jax.ShapeDtypeStruct((NP, D), jnp.float32),
        scratch_types=[
            pltpu.VMEM((bpw,), jnp.int32),
            pltpu.VMEM((bpw, D), jnp.float32),
            pltpu.SemaphoreType.DMA,
        ],
    )
    def k(table_hbm, idx_hbm, out_hbm, idx_v, rows_v, sem):
        wid = lax.axis_index("s") * NC + lax.axis_index("c")
        base = wid * bpw
        pltpu.sync_copy(idx_hbm.at[pl.ds(base, bpw)], idx_v)
        copies = []
        for j in range(0, bpw, 80):
            copies.append(
                pltpu.async_copy(
                    table_hbm.at[idx_v.at[pl.ds(j, 80)]],
                    rows_v.at[pl.ds(j, 80)],
                    sem,
                )
            )
        for c in copies:
            c.wait()
        pltpu.sync_copy(rows_v, out_hbm.at[pl.ds(base, bpw)])

    return k(table, idx_pad)


def _sc_edge_aggregate(hmT, sd, ew):
    mesh = plsc.VectorSubcoreMesh(core_axis_name="c", subcore_axis_name="s")

    @functools.partial(
        pl.kernel,
        mesh=mesh,
        compiler_params=pltpu.CompilerParams(
            use_tc_tiling_on_sc=False, needs_layout_passes=False),
        out_type=jax.ShapeDtypeStruct((D + NW, NP), jnp.float32),
        scratch_types=[
            pltpu.VMEM((NP,), jnp.int32),
            pltpu.VMEM((NP,), jnp.int32),
            pltpu.VMEM((NP,), jnp.float32),
            pltpu.VMEM((NP,), jnp.float32),
            pltpu.VMEM((NP,), jnp.float32),
            pltpu.VMEM((NP,), jnp.float32),
            pltpu.VMEM((NP,), jnp.float32),
            pltpu.VMEM((2, KE), jnp.int32),
            pltpu.VMEM((2, KE), jnp.float32),
            pltpu.SemaphoreType.DMA,
            pltpu.SemaphoreType.DMA,
        ],
    )
    def k(hmT_hbm, sd_hbm, ew_hbm, out_hbm,
          c0, c1, a0, a1, a2, a3, ad, pv2, wv2, sem0, sem1):
        cols = (c0, c1)
        accs = (a0, a1, a2, a3)
        wid = lax.axis_index("s") * NC + lax.axis_index("c")
        zeros16 = jnp.broadcast_to((wid * 0).astype(jnp.float32), (16,))

        for r in range(2):
            pltpu.sync_copy(hmT_hbm.at[2 * wid + r], cols[r])

        @plsc.parallel_loop(0, NP, 16, unroll=4)
        def _(b):
            for r in range(4):
                accs[r][pl.ds(b, 16)] = zeros16
            ad[pl.ds(b, 16)] = zeros16

        sems = (sem0, sem1)

        def issue(ch, slot):
            off = ch * KE
            pltpu.async_copy(sd_hbm.at[pl.ds(off, KE)], pv2.at[slot], sems[slot])
            pltpu.async_copy(ew_hbm.at[pl.ds(off, KE)], wv2.at[slot], sems[slot])

        def drain(slot):
            pltpu.make_async_copy(sd_hbm.at[pl.ds(0, KE)], pv2.at[slot], sems[slot]).wait()
            pltpu.make_async_copy(ew_hbm.at[pl.ds(0, KE)], wv2.at[slot], sems[slot]).wait()

        def consume(slot):
            pv, wv = pv2.at[slot], wv2.at[slot]

            @plsc.parallel_loop(0, KE, 16, unroll=8)
            def _(b):
                p = pv[pl.ds(b, 16)]
                s = jnp.bitwise_and(p, 16383)
                d = jnp.right_shift(p, 14)
                w = wv[pl.ds(b, 16)]
                for q in range(2):
                    pk = plsc.load_gather(cols[q], [s])
                    lo = plsc.bitcast(jnp.left_shift(pk, 16), jnp.float32)
                    hi = plsc.bitcast(jnp.bitwise_and(pk, -65536), jnp.float32)
                    plsc.addupdate_scatter(accs[2 * q], [d], lo * w)
                    plsc.addupdate_scatter(accs[2 * q + 1], [d], hi * w)

        npairs = E // (2 * KE)
        issue(0, 0)

        def pair_body(g, _):
            issue(2 * g + 1, 1)
            drain(0)
            consume(0)

            @pl.when(g + 1 < npairs)
            def _():
                issue(2 * g + 2, 0)
            drain(1)
            consume(1)
            return 0
        lax.fori_loop(0, npairs, pair_body, 0)

        shard = E // NW
        pv0, wv0 = pv2.at[0], wv2.at[0]
        for coff, clen in ((0, 10000),):
            pltpu.sync_copy(sd_hbm.at[pl.ds(wid * shard + coff, clen)],
                            pv0.at[pl.ds(0, clen)])
            pltpu.sync_copy(ew_hbm.at[pl.ds(wid * shard + coff, clen)],
                            wv0.at[pl.ds(0, clen)])

            @plsc.parallel_loop(0, clen, 16, unroll=8)
            def _(b):
                d = jnp.right_shift(pv0[pl.ds(b, 16)], 14)
                plsc.addupdate_scatter(ad, [d], wv0[pl.ds(b, 16)])

        outs = [
            pltpu.async_copy(accs[0], out_hbm.at[2 * wid], sem0),
            pltpu.async_copy(accs[1], out_hbm.at[2 * wid + 64], sem0),
            pltpu.async_copy(accs[2], out_hbm.at[2 * wid + 1], sem0),
            pltpu.async_copy(accs[3], out_hbm.at[2 * wid + 65], sem0),
            pltpu.async_copy(ad, out_hbm.at[D + wid], sem0),
        ]
        for c in outs:
            c.wait()

    return k(hmT, sd, ew)


def _tc_embed_to_messages(x, data2d, W_in, W_msg, src2d, dst2d):
    def body(x_ref, d_ref, wi_ref, wm_ref, s_ref, t_ref, h_ref, hmT_ref, sd_ref):
        m = (d_ref[...] != 0).astype(jnp.float32)
        xb = x_ref[...] * m
        hb = jnp.tanh(jnp.dot(xb, wi_ref[...], preferred_element_type=jnp.float32))
        h_ref[...] = hb
        hm = jnp.dot(hb, wm_ref[...], preferred_element_type=jnp.float32)
        bits = lax.bitcast_convert_type(hm, jnp.int32)
        rne = jnp.right_shift(
            bits + 0x7FFF + jnp.bitwise_and(jnp.right_shift(bits, 16), 1), 16)
        lo = jnp.bitwise_and(rne[:, : H // 2], 0xFFFF)
        hi = jnp.left_shift(rne[:, H // 2 :], 16)
        hmT_ref[...] = jnp.bitwise_or(lo, hi).T
        sd_ref[...] = jnp.bitwise_or(s_ref[...], jnp.left_shift(t_ref[...], 14))

    eb = EB // (NP // RB)
    return pl.pallas_call(
        body,
        grid=(NP // RB,),
        in_specs=[
            pl.BlockSpec((RB, D), lambda j: (j, 0)),
            pl.BlockSpec((RB, 1), lambda j: (j, 0)),
            pl.BlockSpec((D, H), lambda j: (0, 0)),
            pl.BlockSpec((H, H), lambda j: (0, 0)),
            pl.BlockSpec((eb, 160), lambda j: (j, 0)),
            pl.BlockSpec((eb, 160), lambda j: (j, 0)),
        ],
        out_specs=[
            pl.BlockSpec((RB, H), lambda j: (j, 0)),
            pl.BlockSpec((H // 2, RB), lambda j: (0, j)),
            pl.BlockSpec((eb, 160), lambda j: (j, 0)),
        ],
        out_shape=[
            jax.ShapeDtypeStruct((NP, H), jnp.float32),
            jax.ShapeDtypeStruct((H // 2, NP), jnp.int32),
            jax.ShapeDtypeStruct((EB, 160), jnp.int32),
        ],
    )(x, data2d, W_in, W_msg, src2d, dst2d)


def _tc_update(aggT_ext, h, data2d, W_self, b2d):
    def body(a_ref, h_ref, d_ref, ws_ref, b_ref, out_ref, sum_ref, cnt_ref):
        j = pl.program_id(0)
        a = a_ref[...]
        deg = jnp.sum(a[D:, :], axis=0)
        agg = a[:D, :].T
        m = (d_ref[...] != 0).astype(jnp.float32)
        h2 = jnp.tanh(
            agg / (deg[:, None] + 1e-6)
            + jnp.dot(h_ref[...], ws_ref[...], preferred_element_type=jnp.float32)
            + b_ref[...]
        )
        out = h2 * m
        out_ref[...] = out
        ps = jnp.sum(out, axis=0, keepdims=True)
        pc = jnp.sum(m, keepdims=True).reshape(1, 1)

        @pl.when(j == 0)
        def _():
            sum_ref[...] = ps
            cnt_ref[...] = pc

        @pl.when(j != 0)
        def _():
            sum_ref[...] += ps
            cnt_ref[...] += pc

    return pl.pallas_call(
        body,
        grid=(NP // RB,),
        in_specs=[
            pl.BlockSpec((D + NW, RB), lambda j: (0, j)),
            pl.BlockSpec((RB, H), lambda j: (j, 0)),
            pl.BlockSpec((RB, 1), lambda j: (j, 0)),
            pl.BlockSpec((H, H), lambda j: (0, 0)),
            pl.BlockSpec((1, H), lambda j: (0, 0)),
        ],
        out_specs=[
            pl.BlockSpec((RB, H), lambda j: (j, 0)),
            pl.BlockSpec((1, H), lambda j: (0, 0)),
            pl.BlockSpec((1, 1), lambda j: (0, 0)),
        ],
        out_shape=[
            jax.ShapeDtypeStruct((N, H), jnp.float32),
            jax.ShapeDtypeStruct((1, H), jnp.float32),
            jax.ShapeDtypeStruct((1, 1), jnp.float32),
        ],
    )(aggT_ext, h, data2d, W_self, b2d)


def kernel(data, edge_index, edge_weight, embed, W_in, W_msg, W_self, b):
    data_pad = jnp.pad(data.astype(jnp.int32), (0, NP - N))
    data2d = data_pad.reshape(NP, 1)

    ei3 = edge_index.astype(jnp.int32).reshape(2, EB, 160)
    x = _sc_gather_rows(embed, data_pad)
    h, hmT, sd2d = _tc_embed_to_messages(x, data2d, W_in, W_msg, ei3[0], ei3[1])

    aggT_ext = _sc_edge_aggregate(hmT, sd2d.reshape(E), edge_weight)

    out, sum_vec, cnt = _tc_update(aggT_ext, h, data2d, W_self, b.reshape(1, H))

    denom = jnp.maximum(cnt[0, 0], 1.0)
    return (out, sum_vec[0] / denom)


# speedup vs baseline: 1.0518x; 1.0025x over previous
import functools

import jax
import jax.numpy as jnp
from jax import lax
from jax.experimental import pallas as pl
from jax.experimental.pallas import tpu as pltpu
from jax.experimental.pallas import tpu_sc as plsc

N = 10000
NP = 10240
E = 320000
D = 128
H = 128

NC = 2
NS = 16
NW = NC * NS
KE = 10000
RB = 1024
EB = 2000


def _sc_gather_rows(table, idx_pad):
    bpw = NP // NW
    mesh = plsc.VectorSubcoreMesh(core_axis_name="c", subcore_axis_name="s")

    @functools.partial(
        pl.kernel,
        mesh=mesh,
        out_type=jax.ShapeDtypeStruct((NP, D), jnp.float32),
        scratch_types=[
            pltpu.VMEM((bpw,), jnp.int32),
            pltpu.VMEM((bpw, D), jnp.float32),
            pltpu.SemaphoreType.DMA,
        ],
    )
    def k(table_hbm, idx_hbm, out_hbm, idx_v, rows_v, sem):
        wid = lax.axis_index("s") * NC + lax.axis_index("c")
        base = wid * bpw
        pltpu.sync_copy(idx_hbm.at[pl.ds(base, bpw)], idx_v)
        copies = []
        for j in range(0, bpw, 80):
            copies.append(
                pltpu.async_copy(
                    table_hbm.at[idx_v.at[pl.ds(j, 80)]],
                    rows_v.at[pl.ds(j, 80)],
                    sem,
                )
            )
        for c in copies:
            c.wait()
        pltpu.sync_copy(rows_v, out_hbm.at[pl.ds(base, bpw)])

    return k(table, idx_pad)


def _sc_edge_aggregate(hmT, sd, ew):
    mesh = plsc.VectorSubcoreMesh(core_axis_name="c", subcore_axis_name="s")

    @functools.partial(
        pl.kernel,
        mesh=mesh,
        compiler_params=pltpu.CompilerParams(
            use_tc_tiling_on_sc=False, needs_layout_passes=False),
        out_type=jax.ShapeDtypeStruct((D + NW, NP), jnp.float32),
        scratch_types=[
            pltpu.VMEM((NP,), jnp.int32),
            pltpu.VMEM((NP,), jnp.int32),
            pltpu.VMEM((NP,), jnp.float32),
            pltpu.VMEM((NP,), jnp.float32),
            pltpu.VMEM((NP,), jnp.float32),
            pltpu.VMEM((NP,), jnp.float32),
            pltpu.VMEM((NP,), jnp.float32),
            pltpu.VMEM((2, KE), jnp.int32),
            pltpu.VMEM((2, KE), jnp.float32),
            pltpu.SemaphoreType.DMA,
            pltpu.SemaphoreType.DMA,
        ],
    )
    def k(hmT_hbm, sd_hbm, ew_hbm, out_hbm,
          c0, c1, a0, a1, a2, a3, ad, pv2, wv2, sem0, sem1):
        cols = (c0, c1)
        accs = (a0, a1, a2, a3)
        wid = lax.axis_index("s") * NC + lax.axis_index("c")
        zeros16 = jnp.broadcast_to((wid * 0).astype(jnp.float32), (16,))

        for r in range(2):
            pltpu.sync_copy(hmT_hbm.at[2 * wid + r], cols[r])

        @plsc.parallel_loop(0, NP, 16, unroll=4)
        def _(b):
            for r in range(4):
                accs[r][pl.ds(b, 16)] = zeros16
            ad[pl.ds(b, 16)] = zeros16

        sems = (sem0, sem1)

        def issue(ch, slot):
            off = ch * KE
            pltpu.async_copy(sd_hbm.at[pl.ds(off, KE)], pv2.at[slot], sems[slot])
            pltpu.async_copy(ew_hbm.at[pl.ds(off, KE)], wv2.at[slot], sems[slot])

        def drain(slot):
            pltpu.make_async_copy(sd_hbm.at[pl.ds(0, KE)], pv2.at[slot], sems[slot]).wait()
            pltpu.make_async_copy(ew_hbm.at[pl.ds(0, KE)], wv2.at[slot], sems[slot]).wait()

        def consume(slot):
            pv, wv = pv2.at[slot], wv2.at[slot]

            @plsc.parallel_loop(0, KE, 16, unroll=8)
            def _(b):
                p = pv[pl.ds(b, 16)]
                s = jnp.bitwise_and(p, 16383)
                d = jnp.right_shift(p, 14)
                w = wv[pl.ds(b, 16)]
                for q in range(2):
                    pk = plsc.load_gather(cols[q], [s])
                    lo = plsc.bitcast(jnp.left_shift(pk, 16), jnp.float32)
                    hi = plsc.bitcast(jnp.bitwise_and(pk, -65536), jnp.float32)
                    plsc.addupdate_scatter(accs[2 * q], [d], lo * w)
                    plsc.addupdate_scatter(accs[2 * q + 1], [d], hi * w)

        npairs = E // (2 * KE)
        issue(0, 0)

        def pair_body(g, _):
            issue(2 * g + 1, 1)
            drain(0)
            consume(0)

            @pl.when(g + 1 < npairs)
            def _():
                issue(2 * g + 2, 0)
            drain(1)
            consume(1)
            return 0
        lax.fori_loop(0, npairs, pair_body, 0)

        shard = E // NW
        pv0, wv0 = pv2.at[0], wv2.at[0]
        for coff, clen in ((0, 10000),):
            dc0 = pltpu.async_copy(sd_hbm.at[pl.ds(wid * shard + coff, clen)],
                                   pv0.at[pl.ds(0, clen)], sem0)
            dc1 = pltpu.async_copy(ew_hbm.at[pl.ds(wid * shard + coff, clen)],
                                   wv0.at[pl.ds(0, clen)], sem1)
            dc0.wait()
            dc1.wait()

            @plsc.parallel_loop(0, clen, 16, unroll=8)
            def _(b):
                d = jnp.right_shift(pv0[pl.ds(b, 16)], 14)
                plsc.addupdate_scatter(ad, [d], wv0[pl.ds(b, 16)])

        outs = [
            pltpu.async_copy(accs[0], out_hbm.at[2 * wid], sem0),
            pltpu.async_copy(accs[1], out_hbm.at[2 * wid + 64], sem0),
            pltpu.async_copy(accs[2], out_hbm.at[2 * wid + 1], sem0),
            pltpu.async_copy(accs[3], out_hbm.at[2 * wid + 65], sem0),
            pltpu.async_copy(ad, out_hbm.at[D + wid], sem0),
        ]
        for c in outs:
            c.wait()

    return k(hmT, sd, ew)


def _tc_embed_to_messages(x, data2d, W_in, W_msg, src2d, dst2d):
    def body(x_ref, d_ref, wi_ref, wm_ref, s_ref, t_ref, h_ref, hmT_ref, sd_ref):
        m = (d_ref[...] != 0).astype(jnp.float32)
        xb = x_ref[...] * m
        hb = jnp.tanh(jnp.dot(xb, wi_ref[...], preferred_element_type=jnp.float32))
        h_ref[...] = hb
        hm = jnp.dot(hb, wm_ref[...], preferred_element_type=jnp.float32)
        bits = lax.bitcast_convert_type(hm, jnp.int32)
        rne = jnp.right_shift(
            bits + 0x7FFF + jnp.bitwise_and(jnp.right_shift(bits, 16), 1), 16)
        lo = jnp.bitwise_and(rne[:, : H // 2], 0xFFFF)
        hi = jnp.left_shift(rne[:, H // 2 :], 16)
        hmT_ref[...] = jnp.bitwise_or(lo, hi).T
        sd_ref[...] = jnp.bitwise_or(s_ref[...], jnp.left_shift(t_ref[...], 14))

    eb = EB // (NP // RB)
    return pl.pallas_call(
        body,
        grid=(NP // RB,),
        in_specs=[
            pl.BlockSpec((RB, D), lambda j: (j, 0)),
            pl.BlockSpec((RB, 1), lambda j: (j, 0)),
            pl.BlockSpec((D, H), lambda j: (0, 0)),
            pl.BlockSpec((H, H), lambda j: (0, 0)),
            pl.BlockSpec((eb, 160), lambda j: (j, 0)),
            pl.BlockSpec((eb, 160), lambda j: (j, 0)),
        ],
        out_specs=[
            pl.BlockSpec((RB, H), lambda j: (j, 0)),
            pl.BlockSpec((H // 2, RB), lambda j: (0, j)),
            pl.BlockSpec((eb, 160), lambda j: (j, 0)),
        ],
        out_shape=[
            jax.ShapeDtypeStruct((NP, H), jnp.float32),
            jax.ShapeDtypeStruct((H // 2, NP), jnp.int32),
            jax.ShapeDtypeStruct((EB, 160), jnp.int32),
        ],
    )(x, data2d, W_in, W_msg, src2d, dst2d)


def _tc_update(aggT_ext, h, data2d, W_self, b2d):
    def body(a_ref, h_ref, d_ref, ws_ref, b_ref, out_ref, sum_ref, cnt_ref):
        j = pl.program_id(0)
        a = a_ref[...]
        deg = jnp.sum(a[D:, :], axis=0)
        agg = a[:D, :].T
        m = (d_ref[...] != 0).astype(jnp.float32)
        h2 = jnp.tanh(
            agg / (deg[:, None] + 1e-6)
            + jnp.dot(h_ref[...], ws_ref[...], preferred_element_type=jnp.float32)
            + b_ref[...]
        )
        out = h2 * m
        out_ref[...] = out
        ps = jnp.sum(out, axis=0, keepdims=True)
        pc = jnp.sum(m, keepdims=True).reshape(1, 1)

        @pl.when(j == 0)
        def _():
            sum_ref[...] = ps
            cnt_ref[...] = pc

        @pl.when(j != 0)
        def _():
            sum_ref[...] += ps
            cnt_ref[...] += pc

    return pl.pallas_call(
        body,
        grid=(NP // RB,),
        in_specs=[
            pl.BlockSpec((D + NW, RB), lambda j: (0, j)),
            pl.BlockSpec((RB, H), lambda j: (j, 0)),
            pl.BlockSpec((RB, 1), lambda j: (j, 0)),
            pl.BlockSpec((H, H), lambda j: (0, 0)),
            pl.BlockSpec((1, H), lambda j: (0, 0)),
        ],
        out_specs=[
            pl.BlockSpec((RB, H), lambda j: (j, 0)),
            pl.BlockSpec((1, H), lambda j: (0, 0)),
            pl.BlockSpec((1, 1), lambda j: (0, 0)),
        ],
        out_shape=[
            jax.ShapeDtypeStruct((N, H), jnp.float32),
            jax.ShapeDtypeStruct((1, H), jnp.float32),
            jax.ShapeDtypeStruct((1, 1), jnp.float32),
        ],
    )(aggT_ext, h, data2d, W_self, b2d)


def kernel(data, edge_index, edge_weight, embed, W_in, W_msg, W_self, b):
    data_pad = jnp.pad(data.astype(jnp.int32), (0, NP - N))
    data2d = data_pad.reshape(NP, 1)

    ei3 = edge_index.astype(jnp.int32).reshape(2, EB, 160)
    x = _sc_gather_rows(embed, data_pad)
    h, hmT, sd2d = _tc_embed_to_messages(x, data2d, W_in, W_msg, ei3[0], ei3[1])

    aggT_ext = _sc_edge_aggregate(hmT, sd2d.reshape(E), edge_weight)

    out, sum_vec, cnt = _tc_update(aggT_ext, h, data2d, W_self, b.reshape(1, H))

    denom = jnp.maximum(cnt[0, 0], 1.0)
    return (out, sum_vec[0] / denom)
